# Initial kernel scaffold; baseline (speedup 1.0000x reference)
import jax
import jax.numpy as jnp
from jax.experimental import pallas as pl


def kernel(x, pos_embedding):
    raise NotImplementedError("write your pallas kernel here")


# speedup vs baseline: 3.1709x; 3.1709x over previous
import jax
import jax.numpy as jnp
from jax.experimental import pallas as pl


def _copy_body(emb_ref, out_ref):
    out_ref[0] = emb_ref[...]


def kernel(x, pos_embedding):
    batch, seq_len = x.shape
    max_len, d_model = pos_embedding.shape

    blk = 512
    num_blocks = seq_len // blk

    out = pl.pallas_call(
        _copy_body,
        grid=(num_blocks, batch),
        in_specs=[pl.BlockSpec((blk, d_model), lambda i, b: (i, 0))],
        out_specs=pl.BlockSpec((1, blk, d_model), lambda i, b: (b, i, 0)),
        out_shape=jax.ShapeDtypeStruct((batch, seq_len, d_model),
                                       pos_embedding.dtype),
    )(pos_embedding)
    return out


# speedup vs baseline: 3.4283x; 1.0812x over previous
import jax
import jax.numpy as jnp
from jax.experimental import pallas as pl


def _copy_body(emb_ref, out_ref):
    out_ref[0] = emb_ref[...]


def kernel(x, pos_embedding):
    batch, seq_len = x.shape
    max_len, d_model = pos_embedding.shape

    blk = 1024
    num_blocks = seq_len // blk

    out = pl.pallas_call(
        _copy_body,
        grid=(num_blocks, batch),
        in_specs=[pl.BlockSpec((blk, d_model), lambda i, b: (i, 0))],
        out_specs=pl.BlockSpec((1, blk, d_model), lambda i, b: (b, i, 0)),
        out_shape=jax.ShapeDtypeStruct((batch, seq_len, d_model),
                                       pos_embedding.dtype),
    )(pos_embedding)
    return out
